# baseline (speedup 1.0000x reference)
import functools

import jax
import jax.numpy as jnp
from jax.experimental import pallas as pl
from jax.experimental.pallas import tpu as pltpu

BETA = 1.0
K = 5.0
THRESH = 0.5
N_PRIORS = 20000
N_TRUTHS = 256

NP = 20480
ROWS = 8
COLS = NP // ROWS


def _iou_kernel(truths_ref, pri_ref, out_ref):
    cx = pri_ref[0]
    cy = pri_ref[1]
    w = pri_ref[2]
    h = pri_ref[3]
    alpha = pri_ref[4]

    px0 = cx - w * 0.5
    py0 = cy - h * 0.5
    px1 = cx + w * 0.5
    py1 = cy + h * 0.5
    parea = w * h

    iota = (jax.lax.broadcasted_iota(jnp.int32, (ROWS, COLS), 0) * COLS
            + jax.lax.broadcasted_iota(jnp.int32, (ROWS, COLS), 1))
    big = jnp.int32(2 ** 30)

    zeros = jnp.zeros((ROWS, COLS), jnp.float32)

    def body(t, carry):
        bto, matched, last_val = carry
        tx0 = truths_ref[0, t]
        ty0 = truths_ref[1, t]
        tx1 = truths_ref[2, t]
        ty1 = truths_ref[3, t]
        tarea = (tx1 - tx0) * (ty1 - ty0)

        ix = jnp.minimum(px1, tx1) - jnp.maximum(px0, tx0)
        iy = jnp.minimum(py1, ty1) - jnp.maximum(py0, ty0)
        inter = jnp.maximum(ix, 0.0) * jnp.maximum(iy, 0.0)
        union = parea + tarea - inter
        ov = inter / union

        bto = jnp.maximum(bto, ov)
        m = jnp.max(ov)
        a = jnp.min(jnp.where(ov == m, iota, big))
        eq = iota == a
        matched = jnp.where(eq, 1.0, matched)
        last_val = jnp.where(eq, m, last_val)
        return bto, matched, last_val

    bto, matched, last_val = jax.lax.fori_loop(
        0, N_TRUTHS, body, (zeros, zeros, zeros))

    valid = iota < N_PRIORS
    s = jnp.where(valid, jax.nn.sigmoid(alpha), 0.0)
    is_matched = matched > 0.0
    xf = jnp.where(is_matched, K, jnp.where(bto > THRESH, 1.0, 0.0))
    btop = jnp.where(is_matched, last_val, bto)
    msk = xf > 1e-07
    num = jnp.sum(jnp.where(msk, s * xf * btop, 0.0)) + BETA * jnp.sum(s)
    den = jnp.sum(jnp.where(msk, xf, 0.0))
    out_ref[0, 0] = num / den


@jax.jit
def kernel(locs, params, truths):
    pri = jnp.concatenate([locs, params], axis=1).T
    pri = jnp.pad(pri, ((0, 0), (0, NP - N_PRIORS)))
    pri = pri.reshape(5, ROWS, COLS)
    truths_t = truths.T

    out = pl.pallas_call(
        _iou_kernel,
        in_specs=[
            pl.BlockSpec(memory_space=pltpu.SMEM),
            pl.BlockSpec(memory_space=pltpu.VMEM),
        ],
        out_specs=pl.BlockSpec(memory_space=pltpu.SMEM),
        out_shape=jax.ShapeDtypeStruct((1, 1), jnp.float32),
    )(truths_t, pri)
    return out[0, 0]


# speedup vs baseline: 1.7141x; 1.7141x over previous
import jax
import jax.numpy as jnp
from jax.experimental import pallas as pl
from jax.experimental.pallas import tpu as pltpu

BETA = 1.0
K = 5.0
THRESH = 0.5
N_PRIORS = 20000
N_TRUTHS = 256

NP = 20480
TB = 8
NBLK = N_TRUTHS // TB


def _iou_kernel(truths_ref, pri_ref, out_ref):
    cx = pri_ref[0:1]
    cy = pri_ref[1:2]
    w = pri_ref[2:3]
    h = pri_ref[3:4]

    px0 = cx - w * 0.5
    py0 = cy - h * 0.5
    px1 = cx + w * 0.5
    py1 = cy + h * 0.5
    parea = w * h

    lane = jax.lax.broadcasted_iota(jnp.int32, (TB, NP), 1)
    sub = jax.lax.broadcasted_iota(jnp.int32, (TB, 1), 0)
    big = jnp.int32(2 ** 30)

    def body(r, carry):
        bto, tlast, lv = carry
        tb = truths_ref[pl.ds(r * TB, TB), :]
        tx0 = tb[:, 0:1]
        ty0 = tb[:, 1:2]
        tx1 = tb[:, 2:3]
        ty1 = tb[:, 3:4]
        tarea = (tx1 - tx0) * (ty1 - ty0)

        ix = jnp.minimum(px1, tx1) - jnp.maximum(px0, tx0)
        iy = jnp.minimum(py1, ty1) - jnp.maximum(py0, ty0)
        inter = jnp.maximum(ix, 0.0) * jnp.maximum(iy, 0.0)
        union = (parea + tarea) - inter
        ov = inter / union

        bto = jnp.maximum(bto, ov)
        m = jnp.max(ov, axis=1, keepdims=True)
        a = jnp.min(jnp.where(ov == m, lane, big), axis=1, keepdims=True)
        eq = lane == a
        tlast = jnp.where(eq, sub + r * TB, tlast)
        lv = jnp.where(eq, m, lv)
        return bto, tlast, lv

    zf = jnp.zeros((TB, NP), jnp.float32)
    neg = jnp.full((TB, NP), -1, jnp.int32)
    bto, tlast, lv = jax.lax.fori_loop(0, NBLK, body, (zf, neg, zf))

    bto_n = jnp.max(bto, axis=0, keepdims=True)
    tmax = jnp.max(tlast, axis=0, keepdims=True)
    lv_n = jnp.max(jnp.where(tlast == tmax, lv, -1.0), axis=0, keepdims=True)
    matched = tmax >= 0

    alpha = pri_ref[4:5]
    valid = lane[0:1] < N_PRIORS
    s = jnp.where(valid, jax.nn.sigmoid(alpha), 0.0)
    xf = jnp.where(matched, K, jnp.where(bto_n > THRESH, 1.0, 0.0))
    btop = jnp.where(matched, lv_n, bto_n)
    msk = xf > 1e-07
    num = jnp.sum(jnp.where(msk, s * xf * btop, 0.0)) + BETA * jnp.sum(s)
    den = jnp.sum(jnp.where(msk, xf, 0.0))
    out_ref[0, 0] = num / den


@jax.jit
def kernel(locs, params, truths):
    pri = jnp.concatenate([locs, params], axis=1).T
    pri = jnp.pad(pri, ((0, 0), (0, NP - N_PRIORS)))

    out = pl.pallas_call(
        _iou_kernel,
        in_specs=[
            pl.BlockSpec(memory_space=pltpu.VMEM),
            pl.BlockSpec(memory_space=pltpu.VMEM),
        ],
        out_specs=pl.BlockSpec(memory_space=pltpu.SMEM),
        out_shape=jax.ShapeDtypeStruct((1, 1), jnp.float32),
    )(truths, pri)
    return out[0, 0]


# speedup vs baseline: 3.2646x; 1.9045x over previous
import jax
import jax.numpy as jnp
from jax.experimental import pallas as pl
from jax.experimental.pallas import tpu as pltpu

BETA = 1.0
K = 5.0
THRESH = 0.5
N_PRIORS = 20000
N_TRUTHS = 256

NP = 20480
TB = 8
NBLK = N_TRUTHS // TB
CW = 2048
NCHUNK = NP // CW


def _iou_kernel(truths_ref, pri_ref, out_ref,
                px0_ref, py0_ref, px1_ref, py1_ref, pa_ref, bto_ref):
    cx = pri_ref[0:1]
    cy = pri_ref[1:2]
    w = pri_ref[2:3]
    h = pri_ref[3:4]
    px0_ref[...] = jnp.broadcast_to(cx - w * 0.5, (TB, NP))
    py0_ref[...] = jnp.broadcast_to(cy - h * 0.5, (TB, NP))
    px1_ref[...] = jnp.broadcast_to(cx + w * 0.5, (TB, NP))
    py1_ref[...] = jnp.broadcast_to(cy + h * 0.5, (TB, NP))
    pa_ref[...] = jnp.broadcast_to(w * h, (TB, NP))
    bto_ref[...] = jnp.zeros((TB, NP), jnp.float32)

    lane = jax.lax.broadcasted_iota(jnp.int32, (TB, CW), 1)
    sub = jax.lax.broadcasted_iota(jnp.int32, (TB, 1), 0)
    big = jnp.int32(2 ** 30)

    col = jax.lax.broadcasted_iota(jnp.int32, (TB, NBLK), 1)

    def body(r, carry):
        rm_acc, ra_acc = carry
        tb = truths_ref[pl.ds(r * TB, TB), :]
        tx0 = tb[:, 0:1]
        ty0 = tb[:, 1:2]
        tx1 = tb[:, 2:3]
        ty1 = tb[:, 3:4]
        tarea = (tx1 - tx0) * (ty1 - ty0)

        rm = jnp.full((TB, 1), -1.0, jnp.float32)
        ra = jnp.zeros((TB, 1), jnp.int32)
        for c in range(NCHUNK):
            sl = pl.ds(c * CW, CW)
            ix = (jnp.minimum(px1_ref[:, sl], tx1)
                  - jnp.maximum(px0_ref[:, sl], tx0))
            iy = (jnp.minimum(py1_ref[:, sl], ty1)
                  - jnp.maximum(py0_ref[:, sl], ty0))
            inter = jnp.maximum(ix, 0.0) * jnp.maximum(iy, 0.0)
            union = (pa_ref[:, sl] + tarea) - inter
            ov = inter / union

            bto_ref[:, sl] = jnp.maximum(bto_ref[:, sl], ov)
            mc = jnp.max(ov, axis=1, keepdims=True)
            ac = jnp.min(jnp.where(ov == mc, lane + c * CW, big),
                         axis=1, keepdims=True)
            better = mc > rm
            ra = jnp.where(better, ac, ra)
            rm = jnp.maximum(rm, mc)
        hit = col == r
        rm_acc = jnp.where(hit, rm, rm_acc)
        ra_acc = jnp.where(hit, ra, ra_acc)
        return rm_acc, ra_acc

    rm_all, ra_all = jax.lax.fori_loop(
        0, NBLK, body,
        (jnp.zeros((TB, NBLK), jnp.float32), jnp.zeros((TB, NBLK), jnp.int32)))
    alpha = pri_ref[4:5]

    num = 0.0
    den = 0.0
    ssum = 0.0
    for c in range(NCHUNK):
        sl = pl.ds(c * CW, CW)
        lane_c = lane + c * CW
        tl = jnp.full((TB, CW), -1, jnp.int32)
        lv = jnp.zeros((TB, CW), jnp.float32)
        for r in range(NBLK):
            a = ra_all[:, r:r + 1]
            m = rm_all[:, r:r + 1]
            eq = lane_c == a
            tl = jnp.where(eq, sub + r * TB, tl)
            lv = jnp.where(eq, m, lv)
        tmax = jnp.max(tl, axis=0, keepdims=True)
        lvn = jnp.max(jnp.where(tl == tmax, lv, -1.0), axis=0, keepdims=True)
        matched = tmax >= 0
        bton = jnp.max(bto_ref[:, sl], axis=0, keepdims=True)

        valid = lane_c[0:1] < N_PRIORS
        s = jnp.where(valid, jax.nn.sigmoid(alpha[:, c * CW:(c + 1) * CW]), 0.0)
        xf = jnp.where(matched, K, jnp.where(bton > THRESH, 1.0, 0.0))
        btop = jnp.where(matched, lvn, bton)
        msk = xf > 1e-07
        num += jnp.sum(jnp.where(msk, s * xf * btop, 0.0))
        den += jnp.sum(jnp.where(msk, xf, 0.0))
        ssum += jnp.sum(s)

    out_ref[0, 0] = (num + BETA * ssum) / den


@jax.jit
def kernel(locs, params, truths):
    pri = jnp.concatenate([locs, params], axis=1).T
    pri = jnp.pad(pri, ((0, 0), (0, NP - N_PRIORS)))

    out = pl.pallas_call(
        _iou_kernel,
        in_specs=[
            pl.BlockSpec(memory_space=pltpu.VMEM),
            pl.BlockSpec(memory_space=pltpu.VMEM),
        ],
        out_specs=pl.BlockSpec(memory_space=pltpu.SMEM),
        out_shape=jax.ShapeDtypeStruct((1, 1), jnp.float32),
        scratch_shapes=[
            pltpu.VMEM((TB, NP), jnp.float32),
            pltpu.VMEM((TB, NP), jnp.float32),
            pltpu.VMEM((TB, NP), jnp.float32),
            pltpu.VMEM((TB, NP), jnp.float32),
            pltpu.VMEM((TB, NP), jnp.float32),
            pltpu.VMEM((TB, NP), jnp.float32),
        ],
    )(truths, pri)
    return out[0, 0]
